# baseline (speedup 1.0000x reference)
import functools

import jax
import jax.numpy as jnp
from jax import lax
from jax.experimental import pallas as pl
from jax.experimental.pallas import tpu as pltpu
from jax.experimental.pallas import tpu_sc as plsc

H = 2048
W = 2048
C = 8
NW = 32
ROWS_PER_TILE = H // NW
BANDS_PER_TILE = ROWS_PER_TILE // 8
TSLOT = 3
SSLOT = 3
T_BANDS = (0, 2, 4)
S_BANDS = (1, 3, 5, 6, 7)


def _body(in_hbm, cost_hbm, weight_hbm, out_hbm, cost_out_hbm,
          bandbuf, spbuf, smallbuf, t_sems, s_sems):
    in_sems = [t_sems.at[i] for i in range(TSLOT)]
    out_sems = [t_sems.at[TSLOT + i] for i in range(TSLOT)]
    s_in_sems = [s_sems.at[i] for i in range(SSLOT)]
    s_out_sems = [s_sems.at[SSLOT + i] for i in range(SSLOT)]
    wbuf = smallbuf.at[0]
    cbuf = smallbuf.at[1]
    cobuf = smallbuf.at[2]
    wid = lax.axis_index("s") * 2 + lax.axis_index("c")

    pltpu.sync_copy(weight_hbm, wbuf.at[pl.ds(0, C)])
    w = wbuf[...]

    ws = [w[i] for i in range(C)]
    mx = functools.reduce(jnp.maximum, ws)
    e = jnp.exp(w - mx)
    es = [e[i] for i in range(C)]
    s = functools.reduce(lambda a, b: a + b, es)
    p = e / s
    ps = [p[i] for i in range(C)]

    best = ps[0]
    sel = jnp.int32(0)
    for i in range(1, C):
        gt = ps[i] > best
        best = jnp.where(gt, ps[i], best)
        sel = jnp.where(gt, jnp.int32(i), sel)

    row0 = wid * ROWS_PER_TILE
    band0 = wid * BANDS_PER_TILE
    sid = lax.axis_index("s")

    def start_ins(band, dst, sem):
        handles = []
        for r in range(8):
            row = row0 + band * 8 + r
            h = pltpu.async_copy(
                in_hbm.at[row, :, pl.ds(sel, 1), :],
                dst.at[:, pl.ds(r, 1), :],
                sem,
            )
            handles.append(h)
        return handles

    t_bands = T_BANDS
    s_bands = S_BANDS

    def t_buf(k):
        return bandbuf.at[k % TSLOT]

    def s_buf(k):
        return spbuf.at[sid, k % SSLOT]

    ins_t, ins_s, outs_t, outs_s = {}, {}, {}, {}

    def start_t(k):
        ins_t[k] = start_ins(t_bands[k], t_buf(k), in_sems[k % TSLOT])

    def start_s(k):
        ins_s[k] = start_ins(s_bands[k], s_buf(k), s_in_sems[k % SSLOT])

    for k in range(min(TSLOT, len(t_bands))):
        start_t(k)
    for k in range(min(SSLOT, len(s_bands))):
        start_s(k)

    @pl.when(wid == 0)
    def _():
        pltpu.sync_copy(cost_hbm, cbuf.at[pl.ds(0, C)])
        cv = cbuf[...]
        co = cv[0]
        for i in range(1, C):
            co = jnp.where(sel == i, cv[i], co)
        cobuf[...] = jnp.full((16,), co, jnp.float32)
        pltpu.sync_copy(cobuf, cost_out_hbm)

    for k in range(max(len(t_bands), len(s_bands))):
        if k < len(t_bands):
            for h in ins_t[k]:
                h.wait()
            outs_t[k] = pltpu.async_copy(
                t_buf(k), out_hbm.at[band0 + t_bands[k]], out_sems[k % TSLOT]
            )
            if k + TSLOT < len(t_bands):
                outs_t[k].wait()
                start_t(k + TSLOT)
        if k < len(s_bands):
            for h in ins_s[k]:
                h.wait()
            outs_s[k] = pltpu.async_copy(
                s_buf(k), out_hbm.at[band0 + s_bands[k]], s_out_sems[k % SSLOT]
            )
            if k + SSLOT < len(s_bands):
                outs_s[k].wait()
                start_s(k + SSLOT)

    for k in range(max(0, len(t_bands) - TSLOT), len(t_bands)):
        outs_t[k].wait()
    for k in range(max(0, len(s_bands) - SSLOT), len(s_bands)):
        outs_s[k].wait()


def kernel(input, cost, weight):
    in4 = input.reshape(H, 16, 128, C).transpose(0, 1, 3, 2)

    mesh = plsc.VectorSubcoreMesh(core_axis_name="c", subcore_axis_name="s")
    out4, cost_out = pl.kernel(
        _body,
        out_type=[
            jax.ShapeDtypeStruct((H // 8, 16, 8, 128), jnp.float32),
            jax.ShapeDtypeStruct((16,), jnp.float32),
        ],
        mesh=mesh,
        compiler_params=pltpu.CompilerParams(
            needs_layout_passes=False,
            disable_bounds_checks=True,
            skip_device_barrier=True,
        ),
        scratch_types=(
            [
                pltpu.VMEM((TSLOT, 16, 8, 128), jnp.float32),
                pltpu.VMEM_SHARED((16, SSLOT, 16, 8, 128), jnp.float32),
                pltpu.VMEM((3, 16), jnp.float32),
            ]
            + [
                pltpu.SemaphoreType.DMA((2 * TSLOT,)),
                pltpu.SemaphoreType.DMA((2 * SSLOT,)),
            ]
        ),
    )(in4, cost, weight)

    output = out4.transpose(0, 2, 1, 3).reshape(H, W)
    return output, cost_out[0]


# speedup vs baseline: 1.0030x; 1.0030x over previous
import functools

import jax
import jax.numpy as jnp
from jax import lax
from jax.experimental import pallas as pl
from jax.experimental.pallas import tpu as pltpu
from jax.experimental.pallas import tpu_sc as plsc

H = 2048
W = 2048
C = 8
NW = 32
ROWS_PER_TILE = H // NW
BANDS_PER_TILE = ROWS_PER_TILE // 8
TSLOT = 3
SSLOT = 3
T_BANDS = (0, 2, 4, 6, 7)
S_BANDS = (1, 3, 5)


def _body(in_hbm, cost_hbm, weight_hbm, out_hbm, cost_out_hbm,
          bandbuf, spbuf, smallbuf, t_sems, s_sems):
    in_sems = [t_sems.at[i] for i in range(TSLOT)]
    out_sems = [t_sems.at[TSLOT + i] for i in range(TSLOT)]
    s_in_sems = [s_sems.at[i] for i in range(SSLOT)]
    s_out_sems = [s_sems.at[SSLOT + i] for i in range(SSLOT)]
    wbuf = smallbuf.at[0]
    cbuf = smallbuf.at[1]
    cobuf = smallbuf.at[2]
    wid = lax.axis_index("s") * 2 + lax.axis_index("c")

    pltpu.sync_copy(weight_hbm, wbuf.at[pl.ds(0, C)])
    w = wbuf[...]

    ws = [w[i] for i in range(C)]
    mx = functools.reduce(jnp.maximum, ws)
    e = jnp.exp(w - mx)
    es = [e[i] for i in range(C)]
    s = functools.reduce(lambda a, b: a + b, es)
    p = e / s
    ps = [p[i] for i in range(C)]

    best = ps[0]
    sel = jnp.int32(0)
    for i in range(1, C):
        gt = ps[i] > best
        best = jnp.where(gt, ps[i], best)
        sel = jnp.where(gt, jnp.int32(i), sel)

    row0 = wid * ROWS_PER_TILE
    band0 = wid * BANDS_PER_TILE
    sid = lax.axis_index("s")

    def start_ins(band, dst, sem):
        handles = []
        for r in range(8):
            row = row0 + band * 8 + r
            h = pltpu.async_copy(
                in_hbm.at[row, :, pl.ds(sel, 1), :],
                dst.at[:, pl.ds(r, 1), :],
                sem,
            )
            handles.append(h)
        return handles

    t_bands = T_BANDS
    s_bands = S_BANDS

    def t_buf(k):
        return bandbuf.at[k % TSLOT]

    def s_buf(k):
        return spbuf.at[sid, k % SSLOT]

    ins_t, ins_s, outs_t, outs_s = {}, {}, {}, {}

    def start_t(k):
        ins_t[k] = start_ins(t_bands[k], t_buf(k), in_sems[k % TSLOT])

    def start_s(k):
        ins_s[k] = start_ins(s_bands[k], s_buf(k), s_in_sems[k % SSLOT])

    for k in range(min(TSLOT, len(t_bands))):
        start_t(k)
    for k in range(min(SSLOT, len(s_bands))):
        start_s(k)

    @pl.when(wid == 0)
    def _():
        pltpu.sync_copy(cost_hbm, cbuf.at[pl.ds(0, C)])
        cv = cbuf[...]
        co = cv[0]
        for i in range(1, C):
            co = jnp.where(sel == i, cv[i], co)
        cobuf[...] = jnp.full((16,), co, jnp.float32)
        pltpu.sync_copy(cobuf, cost_out_hbm)

    for k in range(max(len(t_bands), len(s_bands))):
        if k < len(t_bands):
            for h in ins_t[k]:
                h.wait()
            outs_t[k] = pltpu.async_copy(
                t_buf(k), out_hbm.at[band0 + t_bands[k]], out_sems[k % TSLOT]
            )
            if k + TSLOT < len(t_bands):
                outs_t[k].wait()
                start_t(k + TSLOT)
        if k < len(s_bands):
            for h in ins_s[k]:
                h.wait()
            outs_s[k] = pltpu.async_copy(
                s_buf(k), out_hbm.at[band0 + s_bands[k]], s_out_sems[k % SSLOT]
            )
            if k + SSLOT < len(s_bands):
                outs_s[k].wait()
                start_s(k + SSLOT)

    for k in range(max(0, len(t_bands) - TSLOT), len(t_bands)):
        outs_t[k].wait()
    for k in range(max(0, len(s_bands) - SSLOT), len(s_bands)):
        outs_s[k].wait()


def kernel(input, cost, weight):
    in4 = input.reshape(H, 16, 128, C).transpose(0, 1, 3, 2)

    mesh = plsc.VectorSubcoreMesh(core_axis_name="c", subcore_axis_name="s")
    out4, cost_out = pl.kernel(
        _body,
        out_type=[
            jax.ShapeDtypeStruct((H // 8, 16, 8, 128), jnp.float32),
            jax.ShapeDtypeStruct((16,), jnp.float32),
        ],
        mesh=mesh,
        compiler_params=pltpu.CompilerParams(
            needs_layout_passes=False,
            disable_bounds_checks=True,
            skip_device_barrier=True,
        ),
        scratch_types=(
            [
                pltpu.VMEM((TSLOT, 16, 8, 128), jnp.float32),
                pltpu.VMEM_SHARED((16, SSLOT, 16, 8, 128), jnp.float32),
                pltpu.VMEM((3, 16), jnp.float32),
            ]
            + [
                pltpu.SemaphoreType.DMA((2 * TSLOT,)),
                pltpu.SemaphoreType.DMA((2 * SSLOT,)),
            ]
        ),
    )(in4, cost, weight)

    output = out4.transpose(0, 2, 1, 3).reshape(H, W)
    return output, cost_out[0]


# speedup vs baseline: 1.0084x; 1.0054x over previous
import functools

import jax
import jax.numpy as jnp
from jax import lax
from jax.experimental import pallas as pl
from jax.experimental.pallas import tpu as pltpu
from jax.experimental.pallas import tpu_sc as plsc

H = 2048
W = 2048
C = 8
NW = 32
ROWS_PER_TILE = H // NW
BANDS_PER_TILE = ROWS_PER_TILE // 8
TSLOT = 3
SSLOT = 3
T_BANDS = (0, 2, 4, 6)
S_BANDS = (1, 3, 5, 7)


def _body(in_hbm, cost_hbm, weight_hbm, out_hbm, cost_out_hbm,
          bandbuf, spbuf, smallbuf, t_sems, s_sems):
    in_sems = [t_sems.at[i] for i in range(TSLOT)]
    out_sems = [t_sems.at[TSLOT + i] for i in range(TSLOT)]
    s_in_sems = [s_sems.at[i] for i in range(SSLOT)]
    s_out_sems = [s_sems.at[SSLOT + i] for i in range(SSLOT)]
    wbuf = smallbuf.at[0]
    cbuf = smallbuf.at[1]
    cobuf = smallbuf.at[2]
    wid = lax.axis_index("s") * 2 + lax.axis_index("c")

    pltpu.sync_copy(weight_hbm, wbuf.at[pl.ds(0, C)])
    w = wbuf[...]

    ws = [w[i] for i in range(C)]
    mx = functools.reduce(jnp.maximum, ws)
    e = jnp.exp(w - mx)
    es = [e[i] for i in range(C)]
    s = functools.reduce(lambda a, b: a + b, es)
    p = e / s
    ps = [p[i] for i in range(C)]

    best = ps[0]
    sel = jnp.int32(0)
    for i in range(1, C):
        gt = ps[i] > best
        best = jnp.where(gt, ps[i], best)
        sel = jnp.where(gt, jnp.int32(i), sel)

    row0 = wid * ROWS_PER_TILE
    band0 = wid * BANDS_PER_TILE
    sid = lax.axis_index("s")

    def start_ins(band, dst, sem):
        handles = []
        for r in range(8):
            row = row0 + band * 8 + r
            h = pltpu.async_copy(
                in_hbm.at[row, :, pl.ds(sel, 1), :],
                dst.at[:, pl.ds(r, 1), :],
                sem,
            )
            handles.append(h)
        return handles

    t_bands = T_BANDS
    s_bands = S_BANDS

    def t_buf(k):
        return bandbuf.at[k % TSLOT]

    def s_buf(k):
        return spbuf.at[sid, k % SSLOT]

    ins_t, ins_s, outs_t, outs_s = {}, {}, {}, {}

    def start_t(k):
        ins_t[k] = start_ins(t_bands[k], t_buf(k), in_sems[k % TSLOT])

    def start_s(k):
        ins_s[k] = start_ins(s_bands[k], s_buf(k), s_in_sems[k % SSLOT])

    for k in range(min(TSLOT, len(t_bands))):
        start_t(k)
    for k in range(min(SSLOT, len(s_bands))):
        start_s(k)

    @pl.when(wid == 0)
    def _():
        pltpu.sync_copy(cost_hbm, cbuf.at[pl.ds(0, C)])
        cv = cbuf[...]
        co = cv[0]
        for i in range(1, C):
            co = jnp.where(sel == i, cv[i], co)
        cobuf[...] = jnp.full((16,), co, jnp.float32)
        pltpu.sync_copy(cobuf, cost_out_hbm)

    for k in range(max(len(t_bands), len(s_bands))):
        if k < len(t_bands):
            for h in ins_t[k]:
                h.wait()
            outs_t[k] = pltpu.async_copy(
                t_buf(k), out_hbm.at[band0 + t_bands[k]], out_sems[k % TSLOT]
            )
            if k + TSLOT < len(t_bands):
                outs_t[k].wait()
                start_t(k + TSLOT)
        if k < len(s_bands):
            for h in ins_s[k]:
                h.wait()
            outs_s[k] = pltpu.async_copy(
                s_buf(k), out_hbm.at[band0 + s_bands[k]], s_out_sems[k % SSLOT]
            )
            if k + SSLOT < len(s_bands):
                outs_s[k].wait()
                start_s(k + SSLOT)

    for k in range(max(0, len(t_bands) - TSLOT), len(t_bands)):
        outs_t[k].wait()
    for k in range(max(0, len(s_bands) - SSLOT), len(s_bands)):
        outs_s[k].wait()


def kernel(input, cost, weight):
    in4 = input.reshape(H, 16, 128, C).transpose(0, 1, 3, 2)

    mesh = plsc.VectorSubcoreMesh(core_axis_name="c", subcore_axis_name="s")
    out4, cost_out = pl.kernel(
        _body,
        out_type=[
            jax.ShapeDtypeStruct((H // 8, 16, 8, 128), jnp.float32),
            jax.ShapeDtypeStruct((16,), jnp.float32),
        ],
        mesh=mesh,
        compiler_params=pltpu.CompilerParams(needs_layout_passes=False),
        scratch_types=(
            [
                pltpu.VMEM((TSLOT, 16, 8, 128), jnp.float32),
                pltpu.VMEM_SHARED((16, SSLOT, 16, 8, 128), jnp.float32),
                pltpu.VMEM((3, 16), jnp.float32),
            ]
            + [
                pltpu.SemaphoreType.DMA((2 * TSLOT,)),
                pltpu.SemaphoreType.DMA((2 * SSLOT,)),
            ]
        ),
    )(in4, cost, weight)

    output = out4.transpose(0, 2, 1, 3).reshape(H, W)
    return output, cost_out[0]
